# baseline (speedup 1.0000x reference)
import functools

import jax
import jax.numpy as jnp
from jax import lax
from jax.experimental import pallas as pl
from jax.experimental.pallas import tpu as pltpu
from jax.experimental.pallas import tpu_sc as plsc

BATCH = 4
SEQ = 2048
TOK = BATCH * SEQ
DIM = 1024
NC, NS, L = 2, 16, 16
NW = NC * NS
RPW = TOK // NW
K = 32
NCH = RPW // K
PAIRS = DIM // 32


def _i32(v):
  return jnp.asarray(v, dtype=jnp.int32)


def _body(W0, W1, W2, idx_hbm, o0, o1, o2, d0, d1, d2, idx_v,
          fbuf0, fbuf1, obuf0, obuf1, gsem0, gsem1,
          osem0, osem1, dsem0, dsem1):
  wid = lax.axis_index("s") * _i32(NC) + lax.axis_index("c")
  base = wid * _i32(RPW)
  pltpu.sync_copy(idx_hbm.at[pl.ds(base, RPW)], idx_v)

  iota = lax.iota(jnp.int32, L)
  even = iota * _i32(2)
  odd = even + _i32(1)

  fbufs = (fbuf0, fbuf1)
  obufs = (obuf0, obuf1)
  gsems = (gsem0, gsem1)
  osems = (osem0, osem1)
  dsems = (dsem0, dsem1)

  def convert(fb, ob):
    @plsc.parallel_loop(_i32(0), _i32(K), _i32(1))
    def conv_row(r):
      rr = jnp.full((L,), r, dtype=jnp.int32)

      def loads(j):
        a = plsc.load_gather(fb, [rr, even + _i32(32 * j)])
        b = plsc.load_gather(fb, [rr, odd + _i32(32 * j)])
        return a, b

      pipe = [loads(0), loads(1)]
      for j in range(PAIRS):
        if j + 2 < PAIRS:
          pipe.append(loads(j + 2))
        a, b = pipe[j]
        z = plsc.pack(a, b, format=plsc.PackFormat.INTERLEAVED)
        ob[r, pl.ds(32 * j, 2 * L)] = z

  def run_table(W, o, d):
    def start_gather(c, b):
      return pltpu.async_copy(
          W.at[idx_v.at[pl.ds(c * _i32(K), K)]], fbufs[b], gsems[b])

    def wait_gather(b):
      pltpu.make_async_copy(W.at[pl.ds(0, K)], fbufs[b], gsems[b]).wait()

    def wait_out(b):
      pltpu.make_async_copy(obufs[b], o.at[pl.ds(0, K)], osems[b]).wait()
      pltpu.make_async_copy(obufs[b], d.at[pl.ds(0, K)], dsems[b]).wait()

    start_gather(_i32(0), 0)

    def pair_body(t, carry):
      for b in range(2):
        c = t * _i32(2) + _i32(b)
        cn = c + _i32(1)

        @pl.when(cn < _i32(NCH))
        def _():
          start_gather(cn, 1 - b)

        wait_gather(b)

        @pl.when(c >= _i32(2))
        def _():
          wait_out(b)

        convert(fbufs[b], obufs[b])
        pltpu.async_copy(
            obufs[b], o.at[pl.ds(base + c * _i32(K), K)], osems[b])
        pltpu.async_copy(
            obufs[b], d.at[pl.ds(base + c * _i32(K), K)], dsems[b])
      return carry

    lax.fori_loop(_i32(0), _i32(NCH // 2), pair_body, _i32(0))
    wait_out(0)
    wait_out(1)

  run_table(W0, o0, d0)
  run_table(W1, o1, d1)
  run_table(W2, o2, d2)


@jax.jit
def _lookup(idx, W0, W1, W2):
  mesh = plsc.VectorSubcoreMesh(core_axis_name="c", subcore_axis_name="s")
  out = pl.kernel(
      _body,
      out_type=[jax.ShapeDtypeStruct((TOK, DIM), jnp.bfloat16)] * 6,
      mesh=mesh,
      scratch_types=[
          pltpu.VMEM((RPW,), jnp.int32),
          pltpu.VMEM((K, DIM), jnp.float32),
          pltpu.VMEM((K, DIM), jnp.float32),
          pltpu.VMEM((K, DIM), jnp.bfloat16),
          pltpu.VMEM((K, DIM), jnp.bfloat16),
          pltpu.SemaphoreType.DMA,
          pltpu.SemaphoreType.DMA,
          pltpu.SemaphoreType.DMA,
          pltpu.SemaphoreType.DMA,
          pltpu.SemaphoreType.DMA,
          pltpu.SemaphoreType.DMA,
      ],
      compiler_params=pltpu.CompilerParams(needs_layout_passes=False),
  )(W0, W1, W2, idx)
  return out


def kernel(inputs, W0, W1, W2):
  idx = inputs.reshape(TOK).astype(jnp.int32)
  outs = _lookup(idx, W0, W1, W2)
  embs = tuple(o.reshape(BATCH, SEQ, DIM) for o in outs)
  return embs


# speedup vs baseline: 1.0496x; 1.0496x over previous
import functools

import jax
import jax.numpy as jnp
from jax import lax
from jax.experimental import pallas as pl
from jax.experimental.pallas import tpu as pltpu
from jax.experimental.pallas import tpu_sc as plsc

BATCH = 4
SEQ = 2048
TOK = BATCH * SEQ
DIM = 1024
NC, NS, L = 2, 16, 16
NW = NC * NS
RPW = TOK // NW
K = 32
NCH = RPW // K
PAIRS = DIM // 32


def _i32(v):
  return jnp.asarray(v, dtype=jnp.int32)


def _body(W0, W1, W2, idx_hbm, o0, o1, o2, d0, d1, d2, idx_v,
          fbuf0, fbuf1, obuf0, obuf1, gsem0, gsem1,
          osem0, osem1, dsem0, dsem1):
  wid = lax.axis_index("s") * _i32(NC) + lax.axis_index("c")
  base = wid * _i32(RPW)
  pltpu.sync_copy(idx_hbm.at[pl.ds(base, RPW)], idx_v)

  iota = lax.iota(jnp.int32, L)
  even = iota * _i32(2)
  odd = even + _i32(1)

  fbufs = (fbuf0, fbuf1)
  obufs = (obuf0, obuf1)
  gsems = (gsem0, gsem1)
  osems = (osem0, osem1)
  dsems = (dsem0, dsem1)

  def convert(fb, ob):
    def conv_row(r, carry):
      rr = jnp.full((L,), r, dtype=jnp.int32)

      def loads(j):
        a = plsc.load_gather(fb, [rr, even + _i32(32 * j)])
        b = plsc.load_gather(fb, [rr, odd + _i32(32 * j)])
        return a, b

      pipe = [loads(0), loads(1), loads(2)]
      for j in range(PAIRS):
        if j + 3 < PAIRS:
          pipe.append(loads(j + 3))
        a, b = pipe[j]
        z = plsc.pack(a, b, format=plsc.PackFormat.INTERLEAVED)
        ob[r, pl.ds(32 * j, 2 * L)] = z
      return carry

    lax.fori_loop(_i32(0), _i32(K), conv_row, _i32(0))

  def run_table(W, o, d):
    def start_gather(c, b):
      return pltpu.async_copy(
          W.at[idx_v.at[pl.ds(c * _i32(K), K)]], fbufs[b], gsems[b])

    def wait_gather(b):
      pltpu.make_async_copy(W.at[pl.ds(0, K)], fbufs[b], gsems[b]).wait()

    def wait_out(b):
      pltpu.make_async_copy(obufs[b], o.at[pl.ds(0, K)], osems[b]).wait()
      pltpu.make_async_copy(obufs[b], d.at[pl.ds(0, K)], dsems[b]).wait()

    start_gather(_i32(0), 0)

    def pair_body(t, carry):
      for b in range(2):
        c = t * _i32(2) + _i32(b)
        cn = c + _i32(1)

        @pl.when(cn < _i32(NCH))
        def _():
          start_gather(cn, 1 - b)

        wait_gather(b)

        @pl.when(c >= _i32(2))
        def _():
          wait_out(b)

        convert(fbufs[b], obufs[b])
        pltpu.async_copy(
            obufs[b], o.at[pl.ds(base + c * _i32(K), K)], osems[b])
        pltpu.async_copy(
            obufs[b], d.at[pl.ds(base + c * _i32(K), K)], dsems[b])
      return carry

    lax.fori_loop(_i32(0), _i32(NCH // 2), pair_body, _i32(0))
    wait_out(0)
    wait_out(1)

  run_table(W0, o0, d0)
  run_table(W1, o1, d1)
  run_table(W2, o2, d2)


@jax.jit
def _lookup(idx, W0, W1, W2):
  mesh = plsc.VectorSubcoreMesh(core_axis_name="c", subcore_axis_name="s")
  out = pl.kernel(
      _body,
      out_type=[jax.ShapeDtypeStruct((TOK, DIM), jnp.bfloat16)] * 6,
      mesh=mesh,
      scratch_types=[
          pltpu.VMEM((RPW,), jnp.int32),
          pltpu.VMEM((K, DIM), jnp.float32),
          pltpu.VMEM((K, DIM), jnp.float32),
          pltpu.VMEM((K, DIM), jnp.bfloat16),
          pltpu.VMEM((K, DIM), jnp.bfloat16),
          pltpu.SemaphoreType.DMA,
          pltpu.SemaphoreType.DMA,
          pltpu.SemaphoreType.DMA,
          pltpu.SemaphoreType.DMA,
          pltpu.SemaphoreType.DMA,
          pltpu.SemaphoreType.DMA,
      ],
      compiler_params=pltpu.CompilerParams(needs_layout_passes=False),
  )(W0, W1, W2, idx)
  return out


def kernel(inputs, W0, W1, W2):
  idx = inputs.reshape(TOK).astype(jnp.int32)
  outs = _lookup(idx, W0, W1, W2)
  embs = tuple(o.reshape(BATCH, SEQ, DIM) for o in outs)
  return embs


# speedup vs baseline: 1.0880x; 1.0366x over previous
import functools

import jax
import jax.numpy as jnp
from jax import lax
from jax.experimental import pallas as pl
from jax.experimental.pallas import tpu as pltpu
from jax.experimental.pallas import tpu_sc as plsc

BATCH = 4
SEQ = 2048
TOK = BATCH * SEQ
DIM = 1024
NC, NS, L = 2, 16, 16
NW = NC * NS
RPW = TOK // NW
K = 32
NCH = RPW // K
PAIRS = DIM // 32


def _i32(v):
  return jnp.asarray(v, dtype=jnp.int32)


def _body(W0, W1, W2, idx_hbm, o0, o1, o2, d0, d1, d2, idx_v,
          fbuf0, fbuf1, obuf0, obuf1, gsem0, gsem1,
          osem0, osem1, dsem0, dsem1):
  wid = lax.axis_index("s") * _i32(NC) + lax.axis_index("c")
  base = wid * _i32(RPW)
  pltpu.sync_copy(idx_hbm.at[pl.ds(base, RPW)], idx_v)

  iota = lax.iota(jnp.int32, L)
  even = iota * _i32(2)
  odd = even + _i32(1)

  fbufs = (fbuf0, fbuf1)
  obufs = (obuf0, obuf1)
  gsems = (gsem0, gsem1)
  osems = (osem0, osem1)
  dsems = (dsem0, dsem1)

  def convert(fb, ob):
    def conv_row(r, carry):
      rr = jnp.full((L,), r, dtype=jnp.int32)

      def loads(j):
        a = plsc.load_gather(fb, [rr, even + _i32(32 * j)])
        b = plsc.load_gather(fb, [rr, odd + _i32(32 * j)])
        return a, b

      pipe = [loads(0), loads(1), loads(2)]
      for j in range(PAIRS):
        if j + 3 < PAIRS:
          pipe.append(loads(j + 3))
        a, b = pipe[j]
        z = plsc.pack(a, b, format=plsc.PackFormat.INTERLEAVED)
        ob[r, pl.ds(32 * j, 2 * L)] = z
      return carry

    lax.fori_loop(_i32(0), _i32(K), conv_row, _i32(0))

  def prime(W):
    pltpu.async_copy(
        W.at[idx_v.at[pl.ds(_i32(0), K)]], fbufs[0], gsems[0])

  def run_table(W, o, d, Wnext):
    def start_gather(c, b):
      return pltpu.async_copy(
          W.at[idx_v.at[pl.ds(c * _i32(K), K)]], fbufs[b], gsems[b])

    def wait_gather(b):
      pltpu.make_async_copy(W.at[pl.ds(0, K)], fbufs[b], gsems[b]).wait()

    def wait_out(b):
      pltpu.make_async_copy(obufs[b], o.at[pl.ds(0, K)], osems[b]).wait()
      pltpu.make_async_copy(obufs[b], d.at[pl.ds(0, K)], dsems[b]).wait()

    def pair_body(t, carry):
      for b in range(2):
        c = t * _i32(2) + _i32(b)
        cn = c + _i32(1)

        @pl.when(cn < _i32(NCH))
        def _():
          start_gather(cn, 1 - b)

        wait_gather(b)

        @pl.when(c >= _i32(2))
        def _():
          wait_out(b)

        convert(fbufs[b], obufs[b])
        pltpu.async_copy(
            obufs[b], o.at[pl.ds(base + c * _i32(K), K)], osems[b])
        pltpu.async_copy(
            obufs[b], d.at[pl.ds(base + c * _i32(K), K)], dsems[b])
      return carry

    lax.fori_loop(_i32(0), _i32(NCH // 2), pair_body, _i32(0))
    if Wnext is not None:
      prime(Wnext)
    wait_out(0)
    wait_out(1)

  prime(W0)
  run_table(W0, o0, d0, W1)
  run_table(W1, o1, d1, W2)
  run_table(W2, o2, d2, None)


@jax.jit
def _lookup(idx, W0, W1, W2):
  mesh = plsc.VectorSubcoreMesh(core_axis_name="c", subcore_axis_name="s")
  out = pl.kernel(
      _body,
      out_type=[jax.ShapeDtypeStruct((TOK, DIM), jnp.bfloat16)] * 6,
      mesh=mesh,
      scratch_types=[
          pltpu.VMEM((RPW,), jnp.int32),
          pltpu.VMEM((K, DIM), jnp.float32),
          pltpu.VMEM((K, DIM), jnp.float32),
          pltpu.VMEM((K, DIM), jnp.bfloat16),
          pltpu.VMEM((K, DIM), jnp.bfloat16),
          pltpu.SemaphoreType.DMA,
          pltpu.SemaphoreType.DMA,
          pltpu.SemaphoreType.DMA,
          pltpu.SemaphoreType.DMA,
          pltpu.SemaphoreType.DMA,
          pltpu.SemaphoreType.DMA,
      ],
      compiler_params=pltpu.CompilerParams(needs_layout_passes=False),
  )(W0, W1, W2, idx)
  return out


def kernel(inputs, W0, W1, W2):
  idx = inputs.reshape(TOK).astype(jnp.int32)
  outs = _lookup(idx, W0, W1, W2)
  embs = tuple(o.reshape(BATCH, SEQ, DIM) for o in outs)
  return embs


# speedup vs baseline: 1.0920x; 1.0036x over previous
import functools

import jax
import jax.numpy as jnp
from jax import lax
from jax.experimental import pallas as pl
from jax.experimental.pallas import tpu as pltpu
from jax.experimental.pallas import tpu_sc as plsc

BATCH = 4
SEQ = 2048
TOK = BATCH * SEQ
DIM = 1024
NC, NS, L = 2, 16, 16
NW = NC * NS
RPW = TOK // NW
K = 32
NCH = RPW // K
PAIRS = DIM // 32


def _i32(v):
  return jnp.asarray(v, dtype=jnp.int32)


def _body(W0, W1, W2, idx_hbm, o0, o1, o2, d0, d1, d2, idx_v,
          fbuf0, fbuf1, obuf0, obuf1, gsem0, gsem1,
          osem0, osem1, dsem0, dsem1):
  wid = lax.axis_index("s") * _i32(NC) + lax.axis_index("c")
  base = wid * _i32(RPW)
  pltpu.sync_copy(idx_hbm.at[pl.ds(base, RPW)], idx_v)

  iota = lax.iota(jnp.int32, L)
  even = iota * _i32(2)
  odd = even + _i32(1)

  fbufs = (fbuf0, fbuf1)
  obufs = (obuf0, obuf1)
  gsems = (gsem0, gsem1)
  osems = (osem0, osem1)
  dsems = (dsem0, dsem1)

  def convert(fb, ob):
    def conv_row(r, carry):
      rr = jnp.full((L,), r, dtype=jnp.int32)

      def loads(j):
        a = plsc.load_gather(fb, [rr, even + _i32(32 * j)])
        b = plsc.load_gather(fb, [rr, odd + _i32(32 * j)])
        return a, b

      pipe = [loads(0), loads(1), loads(2), loads(3)]
      for j in range(PAIRS):
        if j + 4 < PAIRS:
          pipe.append(loads(j + 4))
        a, b = pipe[j]
        z = plsc.pack(a, b, format=plsc.PackFormat.INTERLEAVED)
        ob[r, pl.ds(32 * j, 2 * L)] = z
      return carry

    lax.fori_loop(_i32(0), _i32(K), conv_row, _i32(0))

  def prime(W):
    pltpu.async_copy(
        W.at[idx_v.at[pl.ds(_i32(0), K)]], fbufs[0], gsems[0])

  def run_table(W, o, d, Wnext):
    def start_gather(c, b):
      return pltpu.async_copy(
          W.at[idx_v.at[pl.ds(c * _i32(K), K)]], fbufs[b], gsems[b])

    def wait_gather(b):
      pltpu.make_async_copy(W.at[pl.ds(0, K)], fbufs[b], gsems[b]).wait()

    def wait_out(b):
      pltpu.make_async_copy(obufs[b], o.at[pl.ds(0, K)], osems[b]).wait()
      pltpu.make_async_copy(obufs[b], d.at[pl.ds(0, K)], dsems[b]).wait()

    def pair_body(t, carry):
      for b in range(2):
        c = t * _i32(2) + _i32(b)
        cn = c + _i32(1)

        @pl.when(cn < _i32(NCH))
        def _():
          start_gather(cn, 1 - b)

        wait_gather(b)

        @pl.when(c >= _i32(2))
        def _():
          wait_out(b)

        convert(fbufs[b], obufs[b])
        pltpu.async_copy(
            obufs[b], o.at[pl.ds(base + c * _i32(K), K)], osems[b])
        pltpu.async_copy(
            obufs[b], d.at[pl.ds(base + c * _i32(K), K)], dsems[b])
      return carry

    lax.fori_loop(_i32(0), _i32(NCH // 2), pair_body, _i32(0))
    if Wnext is not None:
      prime(Wnext)
    wait_out(0)
    wait_out(1)

  prime(W0)
  run_table(W0, o0, d0, W1)
  run_table(W1, o1, d1, W2)
  run_table(W2, o2, d2, None)


@jax.jit
def _lookup(idx, W0, W1, W2):
  mesh = plsc.VectorSubcoreMesh(core_axis_name="c", subcore_axis_name="s")
  out = pl.kernel(
      _body,
      out_type=[jax.ShapeDtypeStruct((TOK, DIM), jnp.bfloat16)] * 6,
      mesh=mesh,
      scratch_types=[
          pltpu.VMEM((RPW,), jnp.int32),
          pltpu.VMEM((K, DIM), jnp.float32),
          pltpu.VMEM((K, DIM), jnp.float32),
          pltpu.VMEM((K, DIM), jnp.bfloat16),
          pltpu.VMEM((K, DIM), jnp.bfloat16),
          pltpu.SemaphoreType.DMA,
          pltpu.SemaphoreType.DMA,
          pltpu.SemaphoreType.DMA,
          pltpu.SemaphoreType.DMA,
          pltpu.SemaphoreType.DMA,
          pltpu.SemaphoreType.DMA,
      ],
      compiler_params=pltpu.CompilerParams(needs_layout_passes=False),
  )(W0, W1, W2, idx)
  return out


def kernel(inputs, W0, W1, W2):
  idx = inputs.reshape(TOK).astype(jnp.int32)
  outs = _lookup(idx, W0, W1, W2)
  embs = tuple(o.reshape(BATCH, SEQ, DIM) for o in outs)
  return embs


# speedup vs baseline: 1.0945x; 1.0023x over previous
import jax
import jax.numpy as jnp
from jax import lax
from jax.experimental import pallas as pl
from jax.experimental.pallas import tpu as pltpu
from jax.experimental.pallas import tpu_sc as plsc

BATCH = 4
SEQ = 2048
TOK = BATCH * SEQ
DIM = 1024
NC, NS, L = 2, 16, 16
NW = NC * NS
RPW = TOK // NW
K = 32
NCH = RPW // K
PAIRS = DIM // 32


def _i32(v):
  return jnp.asarray(v, dtype=jnp.int32)


def _body(W0, W1, W2, idx_hbm, o0, o1, o2, d0, d1, d2, idx_v,
          fbuf0, fbuf1, obuf0, obuf1, gsem0, gsem1,
          osem0, osem1, dsem0, dsem1):
  wid = lax.axis_index("s") * _i32(NC) + lax.axis_index("c")
  base = wid * _i32(RPW)
  pltpu.sync_copy(idx_hbm.at[pl.ds(base, RPW)], idx_v)

  iota = lax.iota(jnp.int32, L)
  even = iota * _i32(2)
  odd = even + _i32(1)

  fbufs = (fbuf0, fbuf1)
  obufs = (obuf0, obuf1)
  gsems = (gsem0, gsem1)
  osems = (osem0, osem1)
  dsems = (dsem0, dsem1)

  def convert(fb, ob):
    def conv_row(r, carry):
      rr = jnp.full((L,), r, dtype=jnp.int32)

      def loads(j):
        a = plsc.load_gather(fb, [rr, even + _i32(32 * j)])
        b = plsc.load_gather(fb, [rr, odd + _i32(32 * j)])
        return a, b

      pipe = [loads(0), loads(1), loads(2), loads(3)]
      for j in range(PAIRS):
        if j + 4 < PAIRS:
          pipe.append(loads(j + 4))
        a, b = pipe[j]
        z = plsc.pack(a, b, format=plsc.PackFormat.INTERLEAVED)
        ob[r, pl.ds(32 * j, 2 * L)] = z
      return carry

    lax.fori_loop(_i32(0), _i32(K), conv_row, _i32(0))

  def prime(W):
    pltpu.async_copy(
        W.at[idx_v.at[pl.ds(_i32(0), K)]], fbufs[0], gsems[0])

  def run_table(W, o, d, Wnext):
    def start_gather(c, b):
      return pltpu.async_copy(
          W.at[idx_v.at[pl.ds(c * _i32(K), K)]], fbufs[b], gsems[b])

    def wait_gather(b):
      pltpu.make_async_copy(W.at[pl.ds(0, K)], fbufs[b], gsems[b]).wait()

    def wait_out(b):
      pltpu.make_async_copy(obufs[b], o.at[pl.ds(0, K)], osems[b]).wait()
      pltpu.make_async_copy(obufs[b], d.at[pl.ds(0, K)], dsems[b]).wait()

    def pair_body(t, carry):
      for b in range(2):
        c = t * _i32(2) + _i32(b)
        cn = c + _i32(1)

        @pl.when(cn < _i32(NCH))
        def _():
          start_gather(cn, 1 - b)

        wait_gather(b)

        @pl.when(c >= _i32(2))
        def _():
          wait_out(b)

        convert(fbufs[b], obufs[b])
        pltpu.async_copy(
            obufs[b], o.at[pl.ds(base + c * _i32(K), K)], osems[b])
        pltpu.async_copy(
            obufs[b], d.at[pl.ds(base + c * _i32(K), K)], dsems[b])
      return carry

    lax.fori_loop(_i32(0), _i32(NCH // 2), pair_body, _i32(0))
    if Wnext is not None:
      prime(Wnext)
    wait_out(0)
    wait_out(1)

  prime(W0)
  run_table(W0, o0, d0, W1)
  run_table(W1, o1, d1, W2)
  run_table(W2, o2, d2, None)


@jax.jit
def _lookup(idx, W0, W1, W2):
  mesh = plsc.VectorSubcoreMesh(core_axis_name="c", subcore_axis_name="s")
  out = pl.kernel(
      _body,
      out_type=[jax.ShapeDtypeStruct((TOK, DIM), jnp.bfloat16)] * 6,
      mesh=mesh,
      scratch_types=[
          pltpu.VMEM((RPW,), jnp.int32),
          pltpu.VMEM((K, DIM), jnp.float32),
          pltpu.VMEM((K, DIM), jnp.float32),
          pltpu.VMEM((K, DIM), jnp.bfloat16),
          pltpu.VMEM((K, DIM), jnp.bfloat16),
          pltpu.SemaphoreType.DMA,
          pltpu.SemaphoreType.DMA,
          pltpu.SemaphoreType.DMA,
          pltpu.SemaphoreType.DMA,
          pltpu.SemaphoreType.DMA,
          pltpu.SemaphoreType.DMA,
      ],
      compiler_params=pltpu.CompilerParams(needs_layout_passes=False),
  )(W0, W1, W2, idx)
  return out


def kernel(inputs, W0, W1, W2):
  idx = inputs.reshape(TOK).astype(jnp.int32)
  outs = _lookup(idx, W0, W1, W2)
  embs = tuple(o.reshape(BATCH, SEQ, DIM) for o in outs)
  return embs
